# baseline (speedup 1.0000x reference)
import functools

import jax
import jax.numpy as jnp
from jax import lax
from jax.experimental import pallas as pl
from jax.experimental.pallas import tpu as pltpu
from jax.experimental.pallas import tpu_sc as plsc

_VOCAB = 100000
_EMB = 64
_BATCH = 1024
_VB = 6144


def _make_sc_gather():
    info = plsc.get_sparse_core_info()
    nc, ns = info.num_cores, info.num_subcores
    nw = nc * ns
    bpw = _BATCH // nw
    mesh = plsc.VectorSubcoreMesh(core_axis_name="c", subcore_axis_name="s")

    nchunk = 4
    nbuf = 2

    @functools.partial(
        pl.kernel,
        out_type=jax.ShapeDtypeStruct((_BATCH, _EMB), jnp.float32),
        mesh=mesh,
        scratch_types=[
            pltpu.VMEM((bpw,), jnp.int32),
            pltpu.VMEM((nbuf, nchunk, _EMB, 128), jnp.float32),
            pltpu.VMEM((bpw, _EMB), jnp.float32),
            pltpu.SemaphoreType.DMA,
            pltpu.SemaphoreType.DMA,
        ],
        compiler_params=pltpu.CompilerParams(
            disable_bounds_checks=True, needs_layout_passes=False
        ),
    )
    def sc_gather(table_hbm, idx_hbm, out_hbm, idx_v, tiles_v, rows_v, gsem, osem):
        wid = lax.axis_index("s") * nc + lax.axis_index("c")
        base = wid * bpw
        nround = bpw // nchunk
        pltpu.sync_copy(idx_hbm.at[pl.ds(base, bpw)], idx_v)
        tiles_idx = []
        lanes = []
        for g in range(bpw // 16):
            xv = idx_v[pl.ds(g * 16, 16)]
            tiles_idx.append(lax.shift_right_logical(xv, 7))
            lanes.append(xv & 127)

        def fire(rnd):
            buf = rnd % nbuf
            cps = []
            for j in range(nchunk):
                c = rnd * nchunk + j
                st = pl.multiple_of(tiles_idx[c // 16][c % 16] * 128, 128)
                cps.append(
                    pltpu.async_copy(
                        table_hbm.at[:, pl.ds(st, 128)],
                        tiles_v.at[buf, j],
                        gsem,
                    )
                )
            return cps

        def extract(rnd):
            buf = rnd % nbuf
            bvec = jnp.full((16,), buf, jnp.int32)
            for j in range(nchunk):
                c = rnd * nchunk + j
                jvec = jnp.full((16,), j, jnp.int32)
                lvec = jnp.broadcast_to(lanes[c // 16][c % 16], (16,))
                cvec = jnp.full((16,), c, jnp.int32)
                for r in range(_EMB // 16):
                    rows = lax.iota(jnp.int32, 16) + r * 16
                    v = plsc.load_gather(tiles_v, [bvec, jvec, rows, lvec])
                    plsc.store_scatter(rows_v, [cvec, rows], v)

        inflight = fire(0)
        for rnd in range(nround):
            nxt = fire(rnd + 1) if rnd + 1 < nround else []
            for cp in inflight:
                cp.wait()
            extract(rnd)
            inflight = nxt
        pltpu.async_copy(rows_v, out_hbm.at[pl.ds(base, bpw)], osem).wait()

    return sc_gather


_sc_gather = _make_sc_gather()


def _matmul_body(wt_ref, h_ref, b_ref, out_ref):
    out_ref[...] = (
        lax.dot_general(
            wt_ref[...],
            h_ref[...],
            (((0,), (1,)), ((), ())),
            preferred_element_type=jnp.float32,
        )
        + b_ref[...][:, None]
    )


@jax.jit
def kernel(x, embed_table, W, b):
    tbl_t = embed_table.T
    wt = W.T
    h = _sc_gather(tbl_t, x.astype(jnp.int32))
    logits_t = pl.pallas_call(
        _matmul_body,
        grid=(pl.cdiv(_VOCAB, _VB),),
        in_specs=[
            pl.BlockSpec((_EMB, _VB), lambda i: (0, i)),
            pl.BlockSpec((_BATCH, _EMB), lambda i: (0, 0)),
            pl.BlockSpec((_VB,), lambda i: (i,)),
        ],
        out_specs=pl.BlockSpec((_VB, _BATCH), lambda i: (i, 0)),
        out_shape=jax.ShapeDtypeStruct((_VOCAB, _BATCH), jnp.float32),
        compiler_params=pltpu.CompilerParams(vmem_limit_bytes=100 * 1024 * 1024),
    )(wt, h, b)
    return logits_t.T


# speedup vs baseline: 1.0039x; 1.0039x over previous
import functools

import jax
import jax.numpy as jnp
from jax import lax
from jax.experimental import pallas as pl
from jax.experimental.pallas import tpu as pltpu
from jax.experimental.pallas import tpu_sc as plsc

_VOCAB = 100000
_EMB = 64
_BATCH = 1024
_VB = 4096


def _make_sc_gather():
    info = plsc.get_sparse_core_info()
    nc, ns = info.num_cores, info.num_subcores
    nw = nc * ns
    bpw = _BATCH // nw
    mesh = plsc.VectorSubcoreMesh(core_axis_name="c", subcore_axis_name="s")

    nchunk = 4
    nbuf = 3

    @functools.partial(
        pl.kernel,
        out_type=jax.ShapeDtypeStruct((_BATCH, _EMB), jnp.float32),
        mesh=mesh,
        scratch_types=[
            pltpu.VMEM((bpw,), jnp.int32),
            pltpu.VMEM((nbuf, nchunk, _EMB, 128), jnp.float32),
            pltpu.VMEM((bpw, _EMB), jnp.float32),
            pltpu.SemaphoreType.DMA,
            pltpu.SemaphoreType.DMA,
        ],
        compiler_params=pltpu.CompilerParams(
            disable_bounds_checks=True, needs_layout_passes=False
        ),
    )
    def sc_gather(table_hbm, idx_hbm, out_hbm, idx_v, tiles_v, rows_v, gsem, osem):
        wid = lax.axis_index("s") * nc + lax.axis_index("c")
        base = wid * bpw
        nround = bpw // nchunk
        pltpu.sync_copy(idx_hbm.at[pl.ds(base, bpw)], idx_v)
        tiles_idx = []
        lanes = []
        for g in range(bpw // 16):
            xv = idx_v[pl.ds(g * 16, 16)]
            tiles_idx.append(lax.shift_right_logical(xv, 7))
            lanes.append(xv & 127)

        def fire(rnd):
            buf = rnd % nbuf
            cps = []
            for j in range(nchunk):
                c = rnd * nchunk + j
                st = pl.multiple_of(tiles_idx[c // 16][c % 16] * 128, 128)
                cps.append(
                    pltpu.async_copy(
                        table_hbm.at[:, pl.ds(st, 128)],
                        tiles_v.at[buf, j],
                        gsem,
                    )
                )
            return cps

        def extract(rnd):
            buf = rnd % nbuf
            bvec = jnp.full((16,), buf, jnp.int32)
            for j in range(nchunk):
                c = rnd * nchunk + j
                jvec = jnp.full((16,), j, jnp.int32)
                lvec = jnp.broadcast_to(lanes[c // 16][c % 16], (16,))
                cvec = jnp.full((16,), c, jnp.int32)
                for r in range(_EMB // 16):
                    rows = lax.iota(jnp.int32, 16) + r * 16
                    v = plsc.load_gather(tiles_v, [bvec, jvec, rows, lvec])
                    rows_v[c, pl.ds(r * 16, 16)] = v

        pending = [fire(0), fire(1)]
        for rnd in range(nround):
            nxt = fire(rnd + 2) if rnd + 2 < nround else []
            for cp in pending.pop(0):
                cp.wait()
            extract(rnd)
            if nxt:
                pending.append(nxt)
        pltpu.async_copy(rows_v, out_hbm.at[pl.ds(base, bpw)], osem).wait()

    return sc_gather


_sc_gather = _make_sc_gather()


def _matmul_body(wt_ref, h_ref, b_ref, out_ref):
    out_ref[...] = (
        lax.dot_general(
            wt_ref[...],
            h_ref[...],
            (((0,), (1,)), ((), ())),
            preferred_element_type=jnp.float32,
        )
        + b_ref[...][:, None]
    )


@jax.jit
def kernel(x, embed_table, W, b):
    tbl_t = embed_table.T
    wt = W.T
    h = _sc_gather(tbl_t, x.astype(jnp.int32))
    logits_t = pl.pallas_call(
        _matmul_body,
        grid=(pl.cdiv(_VOCAB, _VB),),
        in_specs=[
            pl.BlockSpec((_EMB, _VB), lambda i: (0, i)),
            pl.BlockSpec((_BATCH, _EMB), lambda i: (0, 0)),
            pl.BlockSpec((_VB,), lambda i: (i,)),
        ],
        out_specs=pl.BlockSpec((_VB, _BATCH), lambda i: (i, 0)),
        out_shape=jax.ShapeDtypeStruct((_VOCAB, _BATCH), jnp.float32),
        compiler_params=pltpu.CompilerParams(vmem_limit_bytes=100 * 1024 * 1024),
    )(wt, h, b)
    return logits_t.T
